# Initial kernel scaffold; baseline (speedup 1.0000x reference)
import jax
import jax.numpy as jnp
from jax.experimental import pallas as pl


def kernel(x, edge_index, weight, W1, W2, W3, W4, W5, W6, W7, W8, W9, b1, b2, b3, b4, b5, b6, b7, b8, b9):
    raise NotImplementedError("write your pallas kernel here")


# speedup vs baseline: 5.6173x; 5.6173x over previous
import functools

import jax
import jax.numpy as jnp
from jax import lax
from jax.experimental import pallas as pl
from jax.experimental.pallas import tpu as pltpu
from jax.experimental.pallas import tpu_sc as plsc

N = 10000
D_IN = 128
H = 128
C = 40

NC = 2
NS = 16
NW = NC * NS
L = 16
B = 128

RPT = 624
ZR = 208

E_RAW = 320000
E_TOT = E_RAW + N
_CHUNKS = -(-E_TOT // (NW * B))
EPW = _CHUNKS * B
E_PAD = NW * EPW


_GDN = lax.GatherDimensionNumbers(
    offset_dims=(), collapsed_slice_dims=(0,), start_index_map=(0,))


def _bcast_lane(vec, lane):
    idx = jnp.full((L,), lane, jnp.int32)
    return lax.gather(vec, idx[:, None], _GDN, (1,),
                      mode=lax.GatherScatterMode.PROMISE_IN_BOUNDS)


def _zero_acc(zero_v, acc, s, d):
    z16 = jnp.zeros((L,), jnp.float32)

    def zrow(r, _):
        for k in range(d // L):
            zero_v[r, pl.ds(k * L, L)] = z16
        return 0

    lax.fori_loop(0, ZR, zrow, 0)
    base = s * RPT
    for i in range(RPT // ZR):
        pltpu.sync_copy(zero_v, acc.at[pl.ds(base + i * ZR, ZR)])

    @pl.when(s == NS - 1)
    def _():
        pltpu.sync_copy(zero_v.at[pl.ds(0, N - NS * RPT)],
                        acc.at[pl.ds(NS * RPT, N - NS * RPT)])


def _copy_out(acc, out_hbm, c, s):
    base = s * RPT
    pltpu.sync_copy(acc.at[pl.ds(base, RPT)],
                    out_hbm.at[pl.ds(c * N + base, RPT)])

    @pl.when(s == NS - 1)
    def _():
        tail = N - NS * RPT
        pltpu.sync_copy(acc.at[pl.ds(NS * RPT, tail)],
                        out_hbm.at[pl.ds(c * N + NS * RPT, tail)])


def _make_sc_agg(d):
    mesh = plsc.VectorSubcoreMesh(core_axis_name="c", subcore_axis_name="s")

    @functools.partial(
        pl.kernel,
        out_type=jax.ShapeDtypeStruct((NC * N, d), jnp.float32),
        mesh=mesh,
        scratch_types=[
            pltpu.VMEM((B,), jnp.int32),
            pltpu.VMEM((B,), jnp.int32),
            pltpu.VMEM((B,), jnp.float32),
            pltpu.VMEM((B, d), jnp.float32),
            pltpu.VMEM((ZR, d), jnp.float32),
            pltpu.VMEM_SHARED((N, d), jnp.float32),
            pltpu.SemaphoreType.DMA,
        ],
    )
    def agg(row_hbm, col_hbm, ew_hbm, x2_hbm, out_hbm,
            row_v, col_v, ew_v, rows_v, zero_v, acc, sem):
        c = lax.axis_index("c")
        s = lax.axis_index("s")
        wid = c * NS + s

        _zero_acc(zero_v, acc, s, d)
        plsc.subcore_barrier()

        ebase = wid * EPW

        def chunk(g, _):
            off = ebase + g * B
            pltpu.sync_copy(row_hbm.at[pl.ds(off, B)], row_v)
            pltpu.sync_copy(col_hbm.at[pl.ds(off, B)], col_v)
            pltpu.sync_copy(ew_hbm.at[pl.ds(off, B)], ew_v)
            pltpu.async_copy(x2_hbm.at[row_v], rows_v, sem).wait()

            def scale_one(j, _):
                vec = ew_v[pl.ds((j // L) * L, L)]
                sc = _bcast_lane(vec, j % L)
                row = rows_v.at[j]
                for k in range(d // L):
                    sl = pl.ds(k * L, L)
                    row[sl] = row[sl] * sc
                return 0

            lax.fori_loop(0, B, scale_one, 0)
            pltpu.sync_copy(rows_v, acc.at[col_v], add=True)
            return 0

        lax.fori_loop(0, _CHUNKS, chunk, 0)

        plsc.subcore_barrier()
        _copy_out(acc, out_hbm, c, s)

    return agg


def _make_sc_deg():
    d = H
    mesh = plsc.VectorSubcoreMesh(core_axis_name="c", subcore_axis_name="s")

    @functools.partial(
        pl.kernel,
        out_type=jax.ShapeDtypeStruct((NC * N, d), jnp.float32),
        mesh=mesh,
        scratch_types=[
            pltpu.VMEM((B,), jnp.int32),
            pltpu.VMEM((B,), jnp.float32),
            pltpu.VMEM((B, d), jnp.float32),
            pltpu.VMEM((ZR, d), jnp.float32),
            pltpu.VMEM_SHARED((N, d), jnp.float32),
        ],
    )
    def deg(col_hbm, ew_hbm, out_hbm, col_v, ew_v, rows_v, zero_v, acc):
        c = lax.axis_index("c")
        s = lax.axis_index("s")
        wid = c * NS + s

        _zero_acc(zero_v, acc, s, d)
        plsc.subcore_barrier()

        ebase = wid * EPW

        def chunk(g, _):
            off = ebase + g * B
            pltpu.sync_copy(col_hbm.at[pl.ds(off, B)], col_v)
            pltpu.sync_copy(ew_hbm.at[pl.ds(off, B)], ew_v)

            def bcast_one(j, _):
                vec = ew_v[pl.ds((j // L) * L, L)]
                bc = _bcast_lane(vec, j % L)
                row = rows_v.at[j]
                for k in range(d // L):
                    row[pl.ds(k * L, L)] = bc
                return 0

            lax.fori_loop(0, B, bcast_one, 0)
            pltpu.sync_copy(rows_v, acc.at[col_v], add=True)
            return 0

        lax.fori_loop(0, _CHUNKS, chunk, 0)

        plsc.subcore_barrier()
        _copy_out(acc, out_hbm, c, s)

    return deg


_R = 2000


def _tc_dinv(p0, p1):
    def body(p0_ref, p1_ref, o_ref):
        deg = p0_ref[...][:, 0:1] + p1_ref[...][:, 0:1]
        o_ref[...] = jnp.where(deg > 0, lax.rsqrt(jnp.where(deg > 0, deg, 1.0)), 0.0)

    return pl.pallas_call(
        body,
        grid=(N // _R,),
        in_specs=[pl.BlockSpec((_R, H), lambda i: (i, 0)),
                  pl.BlockSpec((_R, H), lambda i: (i, 0))],
        out_specs=pl.BlockSpec((_R, 1), lambda i: (i, 0)),
        out_shape=jax.ShapeDtypeStruct((N, 1), jnp.float32),
    )(p0, p1)


def _tc_first(x, W, dinv):
    def body(x_ref, w_ref, dv_ref, o_ref):
        o_ref[...] = dv_ref[...] * jnp.dot(
            x_ref[...], w_ref[...], preferred_element_type=jnp.float32)

    return pl.pallas_call(
        body,
        grid=(N // _R,),
        in_specs=[pl.BlockSpec((_R, D_IN), lambda i: (i, 0)),
                  pl.BlockSpec((D_IN, H), lambda i: (0, 0)),
                  pl.BlockSpec((_R, 1), lambda i: (i, 0))],
        out_specs=pl.BlockSpec((_R, H), lambda i: (i, 0)),
        out_shape=jax.ShapeDtypeStruct((N, H), jnp.float32),
    )(x, W, dinv)


def _tc_mid(p0, p1, dinv, b, W):
    dn = W.shape[1]

    def body(p0_ref, p1_ref, dv_ref, b_ref, w_ref, o_ref):
        dv = dv_ref[...]
        h = dv * (p0_ref[...] + p1_ref[...]) + b_ref[...]
        h = jnp.where(h > 0, h, jnp.exp(h) - 1.0)
        o_ref[...] = dv * jnp.dot(h, w_ref[...], preferred_element_type=jnp.float32)

    return pl.pallas_call(
        body,
        grid=(N // _R,),
        in_specs=[pl.BlockSpec((_R, H), lambda i: (i, 0)),
                  pl.BlockSpec((_R, H), lambda i: (i, 0)),
                  pl.BlockSpec((_R, 1), lambda i: (i, 0)),
                  pl.BlockSpec((1, H), lambda i: (0, 0)),
                  pl.BlockSpec((H, dn), lambda i: (0, 0))],
        out_specs=pl.BlockSpec((_R, dn), lambda i: (i, 0)),
        out_shape=jax.ShapeDtypeStruct((N, dn), jnp.float32),
    )(p0, p1, dinv, b, W)


def _tc_last(p0, p1, dinv, b):
    dp = p0.shape[1]

    def body(p0_ref, p1_ref, dv_ref, b_ref, o_ref):
        t = dv_ref[...] * (p0_ref[...] + p1_ref[...]) + b_ref[...]
        o_ref[...] = t[:, :C]

    return pl.pallas_call(
        body,
        grid=(N // _R,),
        in_specs=[pl.BlockSpec((_R, dp), lambda i: (i, 0)),
                  pl.BlockSpec((_R, dp), lambda i: (i, 0)),
                  pl.BlockSpec((_R, 1), lambda i: (i, 0)),
                  pl.BlockSpec((1, dp), lambda i: (0, 0))],
        out_specs=pl.BlockSpec((_R, C), lambda i: (i, 0)),
        out_shape=jax.ShapeDtypeStruct((N, C), jnp.float32),
    )(p0, p1, dinv, b)


_sc_agg128 = _make_sc_agg(128)
_sc_deg = _make_sc_deg()


def kernel(x, edge_index, weight, W1, W2, W3, W4, W5, W6, W7, W8, W9,
           b1, b2, b3, b4, b5, b6, b7, b8, b9):
    loop = jnp.arange(N, dtype=jnp.int32)
    pad = E_PAD - E_TOT
    row = jnp.concatenate([edge_index[0].astype(jnp.int32), loop,
                           jnp.zeros((pad,), jnp.int32)])
    col = jnp.concatenate([edge_index[1].astype(jnp.int32), loop,
                           jnp.zeros((pad,), jnp.int32)])
    ew = jnp.concatenate([weight.astype(jnp.float32), jnp.ones((N,), jnp.float32),
                          jnp.zeros((pad,), jnp.float32)])

    degp = _sc_deg(col, ew)
    dinv = _tc_dinv(degp[:N], degp[N:])

    Ws = [W2, W3, W4, W5, W6, W7, W8]
    bs = [b1, b2, b3, b4, b5, b6, b7]

    h2 = _tc_first(x, W1, dinv)
    for i in range(7):
        pp = _sc_agg128(row, col, ew, h2)
        h2 = _tc_mid(pp[:N], pp[N:], dinv, bs[i].reshape(1, H), Ws[i])
    pp = _sc_agg128(row, col, ew, h2)
    W9p = jnp.pad(W9, ((0, 0), (0, H - C)))
    h2 = _tc_mid(pp[:N], pp[N:], dinv, b8.reshape(1, H), W9p)

    pp = _sc_agg128(row, col, ew, h2)
    b9p = jnp.pad(b9, (0, H - C)).reshape(1, H)
    return _tc_last(pp[:N], pp[N:], dinv, b9p)


# speedup vs baseline: 5.6659x; 1.0086x over previous
import functools

import jax
import jax.numpy as jnp
from jax import lax
from jax.experimental import pallas as pl
from jax.experimental.pallas import tpu as pltpu
from jax.experimental.pallas import tpu_sc as plsc

N = 10000
D_IN = 128
H = 128
C = 40

NC = 2
NS = 16
NW = NC * NS
L = 16
B = 128

RPT = 624
ZR = 104

E_RAW = 320000
E_TOT = E_RAW + N
_CHUNKS = 2 * (-(-E_TOT // (NW * B * 2)))
EPW = _CHUNKS * B
E_PAD = NW * EPW


_GDN = lax.GatherDimensionNumbers(
    offset_dims=(), collapsed_slice_dims=(0,), start_index_map=(0,))


def _bcast_lane(vec, lane):
    idx = jnp.full((L,), lane, jnp.int32)
    return lax.gather(vec, idx[:, None], _GDN, (1,),
                      mode=lax.GatherScatterMode.PROMISE_IN_BOUNDS)


def _zero_acc(zero_v, acc, s, d):
    z16 = jnp.zeros((L,), jnp.float32)

    def zrow(r, _):
        for k in range(d // L):
            zero_v[r, pl.ds(k * L, L)] = z16
        return 0

    lax.fori_loop(0, ZR, zrow, 0)
    base = s * RPT
    for i in range(RPT // ZR):
        pltpu.sync_copy(zero_v, acc.at[pl.ds(base + i * ZR, ZR)])

    @pl.when(s == NS - 1)
    def _():
        pltpu.sync_copy(zero_v.at[pl.ds(0, N - NS * RPT)],
                        acc.at[pl.ds(NS * RPT, N - NS * RPT)])


def _copy_out(acc, out_hbm, c, s):
    base = s * RPT
    pltpu.sync_copy(acc.at[pl.ds(base, RPT)],
                    out_hbm.at[pl.ds(c * N + base, RPT)])

    @pl.when(s == NS - 1)
    def _():
        tail = N - NS * RPT
        pltpu.sync_copy(acc.at[pl.ds(NS * RPT, tail)],
                        out_hbm.at[pl.ds(c * N + NS * RPT, tail)])


def _make_sc_agg(d):
    mesh = plsc.VectorSubcoreMesh(core_axis_name="c", subcore_axis_name="s")

    def _scale(rows_v, ew_v):
        def grp(m, _):
            vec = ew_v[pl.ds(m * L, L)]
            for l in range(L):
                sc = _bcast_lane(vec, l)
                row = rows_v.at[m * L + l]
                for k in range(d // L):
                    sl = pl.ds(k * L, L)
                    row[sl] = row[sl] * sc
            return 0

        lax.fori_loop(0, B // L, grp, 0)

    @functools.partial(
        pl.kernel,
        out_type=jax.ShapeDtypeStruct((NC * N, d), jnp.float32),
        mesh=mesh,
        scratch_types=[
            pltpu.VMEM((B,), jnp.int32),
            pltpu.VMEM((B,), jnp.int32),
            pltpu.VMEM((B,), jnp.float32),
            pltpu.VMEM((B,), jnp.int32),
            pltpu.VMEM((B,), jnp.int32),
            pltpu.VMEM((B,), jnp.float32),
            pltpu.VMEM((B, d), jnp.float32),
            pltpu.VMEM((B, d), jnp.float32),
            pltpu.VMEM((ZR, d), jnp.float32),
            pltpu.VMEM_SHARED((N, d), jnp.float32),
            pltpu.SemaphoreType.DMA,
            pltpu.SemaphoreType.DMA,
            pltpu.SemaphoreType.DMA,
            pltpu.SemaphoreType.DMA,
        ],
    )
    def agg(row_hbm, col_hbm, ew_hbm, x2_hbm, out_hbm,
            rowA, colA, ewA, rowB, colB, ewB, rowsA, rowsB, zero_v, acc,
            semiA, semiB, semgA, semgB):
        c = lax.axis_index("c")
        s = lax.axis_index("s")
        wid = c * NS + s

        _zero_acc(zero_v, acc, s, d)
        plsc.subcore_barrier()

        ebase = wid * EPW
        slots = ((rowA, colA, ewA),
                 (rowB, colB, ewB))

        def idx_start(g, slot, sem):
            off = ebase + g * B
            for src_hbm, dst in zip((row_hbm, col_hbm, ew_hbm), slot):
                pltpu.async_copy(src_hbm.at[pl.ds(off, B)], dst, sem)

        def idx_wait(g, slot, sem):
            off = ebase + g * B
            for src_hbm, dst in zip((row_hbm, col_hbm, ew_hbm), slot):
                pltpu.make_async_copy(src_hbm.at[pl.ds(off, B)], dst, sem).wait()

        for src_hbm, dst in zip((row_hbm, col_hbm, ew_hbm), slots[0]):
            pltpu.sync_copy(src_hbm.at[pl.ds(ebase, B)], dst)
        pltpu.async_copy(x2_hbm.at[rowA], rowsA, semgA)

        def pair(i, _):
            g0 = i * 2
            idx_start(g0 + 1, slots[1], semiB)
            pltpu.make_async_copy(x2_hbm.at[rowA], rowsA, semgA).wait()
            _scale(rowsA, ewA)
            idx_wait(g0 + 1, slots[1], semiB)
            pltpu.async_copy(x2_hbm.at[rowB], rowsB, semgB)
            pltpu.sync_copy(rowsA, acc.at[colA], add=True)

            @pl.when(g0 + 2 < _CHUNKS)
            def _():
                idx_start(g0 + 2, slots[0], semiA)

            pltpu.make_async_copy(x2_hbm.at[rowB], rowsB, semgB).wait()
            _scale(rowsB, ewB)

            @pl.when(g0 + 2 < _CHUNKS)
            def _():
                idx_wait(g0 + 2, slots[0], semiA)
                pltpu.async_copy(x2_hbm.at[rowA], rowsA, semgA)

            pltpu.sync_copy(rowsB, acc.at[colB], add=True)
            return 0

        lax.fori_loop(0, _CHUNKS // 2, pair, 0)

        plsc.subcore_barrier()
        _copy_out(acc, out_hbm, c, s)

    return agg


def _make_sc_deg():
    d = H
    mesh = plsc.VectorSubcoreMesh(core_axis_name="c", subcore_axis_name="s")

    @functools.partial(
        pl.kernel,
        out_type=jax.ShapeDtypeStruct((NC * N, d), jnp.float32),
        mesh=mesh,
        scratch_types=[
            pltpu.VMEM((B,), jnp.int32),
            pltpu.VMEM((B,), jnp.float32),
            pltpu.VMEM((B, d), jnp.float32),
            pltpu.VMEM((ZR, d), jnp.float32),
            pltpu.VMEM_SHARED((N, d), jnp.float32),
        ],
    )
    def deg(col_hbm, ew_hbm, out_hbm, col_v, ew_v, rows_v, zero_v, acc):
        c = lax.axis_index("c")
        s = lax.axis_index("s")
        wid = c * NS + s

        _zero_acc(zero_v, acc, s, d)
        plsc.subcore_barrier()

        ebase = wid * EPW

        def chunk(g, _):
            off = ebase + g * B
            pltpu.sync_copy(col_hbm.at[pl.ds(off, B)], col_v)
            pltpu.sync_copy(ew_hbm.at[pl.ds(off, B)], ew_v)

            def bcast_one(j, _):
                vec = ew_v[pl.ds((j // L) * L, L)]
                bc = _bcast_lane(vec, j % L)
                rows_v.at[j][pl.ds(0, L)] = bc
                return 0

            lax.fori_loop(0, B, bcast_one, 0)
            pltpu.sync_copy(rows_v, acc.at[col_v], add=True)
            return 0

        lax.fori_loop(0, _CHUNKS, chunk, 0)

        plsc.subcore_barrier()
        _copy_out(acc, out_hbm, c, s)

    return deg


_R = 2000


def _tc_dinv(p0, p1):
    def body(p0_ref, p1_ref, o_ref):
        deg = p0_ref[...][:, 0:1] + p1_ref[...][:, 0:1]
        o_ref[...] = jnp.where(deg > 0, lax.rsqrt(jnp.where(deg > 0, deg, 1.0)), 0.0)

    return pl.pallas_call(
        body,
        grid=(N // _R,),
        in_specs=[pl.BlockSpec((_R, H), lambda i: (i, 0)),
                  pl.BlockSpec((_R, H), lambda i: (i, 0))],
        out_specs=pl.BlockSpec((_R, 1), lambda i: (i, 0)),
        out_shape=jax.ShapeDtypeStruct((N, 1), jnp.float32),
    )(p0, p1)


def _tc_first(x, W, dinv):
    def body(x_ref, w_ref, dv_ref, o_ref):
        o_ref[...] = dv_ref[...] * jnp.dot(
            x_ref[...], w_ref[...], preferred_element_type=jnp.float32)

    return pl.pallas_call(
        body,
        grid=(N // _R,),
        in_specs=[pl.BlockSpec((_R, D_IN), lambda i: (i, 0)),
                  pl.BlockSpec((D_IN, H), lambda i: (0, 0)),
                  pl.BlockSpec((_R, 1), lambda i: (i, 0))],
        out_specs=pl.BlockSpec((_R, H), lambda i: (i, 0)),
        out_shape=jax.ShapeDtypeStruct((N, H), jnp.float32),
    )(x, W, dinv)


def _tc_mid(p0, p1, dinv, b, W):
    dn = W.shape[1]

    def body(p0_ref, p1_ref, dv_ref, b_ref, w_ref, o_ref):
        dv = dv_ref[...]
        h = dv * (p0_ref[...] + p1_ref[...]) + b_ref[...]
        h = jnp.where(h > 0, h, jnp.exp(h) - 1.0)
        o_ref[...] = dv * jnp.dot(h, w_ref[...], preferred_element_type=jnp.float32)

    return pl.pallas_call(
        body,
        grid=(N // _R,),
        in_specs=[pl.BlockSpec((_R, H), lambda i: (i, 0)),
                  pl.BlockSpec((_R, H), lambda i: (i, 0)),
                  pl.BlockSpec((_R, 1), lambda i: (i, 0)),
                  pl.BlockSpec((1, H), lambda i: (0, 0)),
                  pl.BlockSpec((H, dn), lambda i: (0, 0))],
        out_specs=pl.BlockSpec((_R, dn), lambda i: (i, 0)),
        out_shape=jax.ShapeDtypeStruct((N, dn), jnp.float32),
    )(p0, p1, dinv, b, W)


def _tc_last(p0, p1, dinv, b):
    dp = p0.shape[1]

    def body(p0_ref, p1_ref, dv_ref, b_ref, o_ref):
        t = dv_ref[...] * (p0_ref[...] + p1_ref[...]) + b_ref[...]
        o_ref[...] = t[:, :C]

    return pl.pallas_call(
        body,
        grid=(N // _R,),
        in_specs=[pl.BlockSpec((_R, dp), lambda i: (i, 0)),
                  pl.BlockSpec((_R, dp), lambda i: (i, 0)),
                  pl.BlockSpec((_R, 1), lambda i: (i, 0)),
                  pl.BlockSpec((1, dp), lambda i: (0, 0))],
        out_specs=pl.BlockSpec((_R, C), lambda i: (i, 0)),
        out_shape=jax.ShapeDtypeStruct((N, C), jnp.float32),
    )(p0, p1, dinv, b)


_sc_agg128 = _make_sc_agg(128)
_sc_deg = _make_sc_deg()


def kernel(x, edge_index, weight, W1, W2, W3, W4, W5, W6, W7, W8, W9,
           b1, b2, b3, b4, b5, b6, b7, b8, b9):
    loop = jnp.arange(N, dtype=jnp.int32)
    pad = E_PAD - E_TOT
    row = jnp.concatenate([edge_index[0].astype(jnp.int32), loop,
                           jnp.zeros((pad,), jnp.int32)])
    col = jnp.concatenate([edge_index[1].astype(jnp.int32), loop,
                           jnp.zeros((pad,), jnp.int32)])
    ew = jnp.concatenate([weight.astype(jnp.float32), jnp.ones((N,), jnp.float32),
                          jnp.zeros((pad,), jnp.float32)])

    degp = _sc_deg(col, ew)
    dinv = _tc_dinv(degp[:N], degp[N:])

    Ws = [W2, W3, W4, W5, W6, W7, W8]
    bs = [b1, b2, b3, b4, b5, b6, b7]

    h2 = _tc_first(x, W1, dinv)
    for i in range(7):
        pp = _sc_agg128(row, col, ew, h2)
        h2 = _tc_mid(pp[:N], pp[N:], dinv, bs[i].reshape(1, H), Ws[i])
    pp = _sc_agg128(row, col, ew, h2)
    W9p = jnp.pad(W9, ((0, 0), (0, H - C)))
    h2 = _tc_mid(pp[:N], pp[N:], dinv, b8.reshape(1, H), W9p)

    pp = _sc_agg128(row, col, ew, h2)
    b9p = jnp.pad(b9, (0, H - C)).reshape(1, H)
    return _tc_last(pp[:N], pp[N:], dinv, b9p)
